# baseline (speedup 1.0000x reference)
import functools

import jax
import jax.numpy as jnp
from jax import lax
from jax.experimental import pallas as pl
from jax.experimental.pallas import tpu as pltpu
from jax.experimental.pallas import tpu_sc as plsc

NC = 2
NS = 16
NT = NC * NS
CHUNK = 128
BIG = 256
IDXB = 8


def _mesh():
    return plsc.VectorSubcoreMesh(
        core_axis_name="c", subcore_axis_name="s", num_cores=NC, num_subcores=NS
    )


def _sc_degree(dst_chunks, ones_cols, n_acc, cpt):
    rps = n_acc // NS
    DW = 16

    @functools.partial(
        pl.kernel,
        out_type=jax.ShapeDtypeStruct((NC, n_acc, DW), jnp.float32),
        mesh=_mesh(),
        scratch_types=[
            pltpu.VMEM((cpt, CHUNK), jnp.int32),
            pltpu.VMEM((CHUNK, DW), jnp.float32),
            pltpu.VMEM_SHARED((n_acc, DW), jnp.float32),
        ],
        compiler_params=pltpu.CompilerParams(use_tc_tiling_on_sc=False),
    )
    def deg_kernel(dst_hbm, ones_hbm, out_hbm, dst_v, ones_v, acc):
        c = lax.axis_index("c")
        s = lax.axis_index("s")
        tile = c * NS + s
        pltpu.sync_copy(dst_hbm.at[pl.ds(tile * cpt, cpt)], dst_v)
        pltpu.sync_copy(ones_hbm.at[pl.ds(0, CHUNK)], ones_v)
        pltpu.sync_copy(
            ones_hbm.at[pl.ds(s * rps, rps)], acc.at[pl.ds(s * rps, rps)]
        )
        plsc.subcore_barrier()

        @pl.loop(0, cpt)
        def _(j):
            pltpu.sync_copy(ones_v, acc.at[dst_v.at[j]], add=True)

        plsc.subcore_barrier()
        pltpu.sync_copy(
            acc.at[pl.ds(s * rps, rps)], out_hbm.at[c].at[pl.ds(s * rps, rps)]
        )

    return deg_kernel(dst_chunks, ones_cols)


def _sc_aggregate(g_a, g_b, src_big, dst_big, n_acc, dh, bpt):
    rps = n_acc // NS

    @functools.partial(
        pl.kernel,
        out_type=jax.ShapeDtypeStruct((NC, n_acc, dh), jnp.float32),
        mesh=_mesh(),
        scratch_types=[
            pltpu.VMEM((IDXB * BIG,), jnp.int32),
            pltpu.VMEM((IDXB * 2, CHUNK), jnp.int32),
            pltpu.VMEM((BIG, dh), jnp.float32),
            pltpu.VMEM((BIG, dh), jnp.float32),
            pltpu.VMEM_SHARED((n_acc, dh), jnp.float32),
            pltpu.SemaphoreType.DMA,
            pltpu.SemaphoreType.DMA,
            pltpu.SemaphoreType.DMA,
            pltpu.SemaphoreType.DMA,
        ],
        compiler_params=pltpu.CompilerParams(use_tc_tiling_on_sc=False),
    )
    def agg_kernel(
        ga_hbm, gb_hbm, src_hbm, dst_hbm, out_hbm, src_b, dst_b, rows0, rows1,
        acc, sem0, sem1, ssem0, ssem1,
    ):
        c = lax.axis_index("c")
        s = lax.axis_index("s")

        def run(tab_hbm):
            pltpu.sync_copy(
                tab_hbm.at[pl.ds(s * rps, rps)], acc.at[pl.ds(s * rps, rps)]
            )
            plsc.subcore_barrier()

            @pl.loop(0, bpt // IDXB)
            def _(blk):
                base = s * bpt + blk * IDXB
                pltpu.sync_copy(
                    src_hbm.at[pl.ds(base * BIG, IDXB * BIG)], src_b
                )
                pltpu.sync_copy(dst_hbm.at[pl.ds(base * 2, IDXB * 2)], dst_b)
                pltpu.async_copy(
                    tab_hbm.at[src_b.at[pl.ds(0, BIG)]], rows0, sem0
                )

                def wait_scats(rows, ssem):
                    pltpu.make_async_copy(
                        rows.at[pl.ds(0, CHUNK)], acc.at[dst_b.at[0]], ssem
                    ).wait()
                    pltpu.make_async_copy(
                        rows.at[pl.ds(CHUNK, CHUNK)], acc.at[dst_b.at[0]], ssem
                    ).wait()

                @pl.loop(0, IDXB, step=2)
                def _(j):
                    pltpu.make_async_copy(
                        tab_hbm.at[src_b.at[pl.ds(j * BIG, BIG)]], rows0, sem0
                    ).wait()

                    @pl.when(j > 0)
                    def _():
                        wait_scats(rows1, ssem1)

                    pltpu.async_copy(
                        tab_hbm.at[src_b.at[pl.ds((j + 1) * BIG, BIG)]],
                        rows1, sem1,
                    )
                    pltpu.async_copy(
                        rows0.at[pl.ds(0, CHUNK)], acc.at[dst_b.at[2 * j]],
                        ssem0, add=True,
                    )
                    pltpu.async_copy(
                        rows0.at[pl.ds(CHUNK, CHUNK)],
                        acc.at[dst_b.at[2 * j + 1]], ssem0, add=True,
                    )
                    pltpu.make_async_copy(
                        tab_hbm.at[src_b.at[pl.ds(0, BIG)]], rows1, sem1
                    ).wait()

                    @pl.when(j + 2 < IDXB)
                    def _():
                        wait_scats(rows0, ssem0)
                        pltpu.async_copy(
                            tab_hbm.at[src_b.at[pl.ds((j + 2) * BIG, BIG)]],
                            rows0, sem0,
                        )

                    pltpu.async_copy(
                        rows1.at[pl.ds(0, CHUNK)],
                        acc.at[dst_b.at[2 * j + 2]], ssem1, add=True,
                    )
                    pltpu.async_copy(
                        rows1.at[pl.ds(CHUNK, CHUNK)],
                        acc.at[dst_b.at[2 * j + 3]], ssem1, add=True,
                    )

                wait_scats(rows0, ssem0)
                wait_scats(rows1, ssem1)

            plsc.subcore_barrier()
            pltpu.sync_copy(
                acc.at[pl.ds(s * rps, rps)],
                out_hbm.at[c].at[pl.ds(s * rps, rps)],
            )

        @pl.when(c == 0)
        def _():
            run(ga_hbm)

        @pl.when(c == 1)
        def _():
            run(gb_hbm)

    return agg_kernel(g_a, g_b, src_big, dst_big)


def _dinv(degp_ref):
    deg = degp_ref[0, :, 0:1] + degp_ref[1, :, 0:1] - 1.0
    return lax.rsqrt(jnp.maximum(deg, 1.0))


_DOT = dict(preferred_element_type=jnp.float32, precision=lax.Precision.HIGHEST)


def _tc_pre(x, w, degp, n_acc):
    n, _ = x.shape
    d = w.shape[1]
    dh = d // 2

    def body(x_ref, w_ref, degp_ref, ga_ref, gb_ref, dinv_ref):
        dinv = _dinv(degp_ref)
        h = jnp.dot(x_ref[...], w_ref[...], **_DOT)
        g = dinv[:n] * h
        pad = jnp.zeros((n_acc - n, dh), jnp.float32)
        ga_ref[...] = jnp.concatenate([g[:, :dh], pad], axis=0)
        gb_ref[...] = jnp.concatenate([g[:, dh:], pad], axis=0)
        dinv_ref[...] = jnp.broadcast_to(dinv, (n_acc, 8))

    return pl.pallas_call(
        body,
        out_shape=[
            jax.ShapeDtypeStruct((n_acc, dh), jnp.float32),
            jax.ShapeDtypeStruct((n_acc, dh), jnp.float32),
            jax.ShapeDtypeStruct((n_acc, 8), jnp.float32),
        ],
    )(x, w, degp)


def _tc_mid(dinv8, part, b, w_next):
    n_acc = part.shape[1]
    d = w_next.shape[1]
    dh = d // 2

    def body(dinv_ref, p_ref, b_ref, w_ref, ga_ref, gb_ref):
        dinv = dinv_ref[:, 0:1]
        agg = jnp.concatenate([p_ref[0], p_ref[1]], axis=1)
        x_next = jnp.maximum(dinv * agg + b_ref[...], 0.0)
        g = dinv * jnp.dot(x_next, w_ref[...], **_DOT)
        ga_ref[...] = g[:, :dh]
        gb_ref[...] = g[:, dh:]

    return pl.pallas_call(
        body,
        out_shape=[
            jax.ShapeDtypeStruct((n_acc, dh), jnp.float32),
            jax.ShapeDtypeStruct((n_acc, dh), jnp.float32),
        ],
    )(dinv8, part, b, w_next)


def _tc_final(dinv8, part, b, n):
    n_acc = part.shape[1]
    d = part.shape[2] * 2

    def body(dinv_ref, p_ref, b_ref, o_ref):
        dinv = dinv_ref[:, 0:1]
        agg = jnp.concatenate([p_ref[0], p_ref[1]], axis=1)
        o_ref[...] = (dinv * agg + b_ref[...])[:n]

    return pl.pallas_call(
        body, out_shape=jax.ShapeDtypeStruct((n, d), jnp.float32)
    )(dinv8, part, b)


def kernel(x, edge_index, W1, b1, W2, b2, W3, b3):
    n = x.shape[0]
    e = edge_index.shape[1]
    d = W1.shape[1]
    dh = d // 2

    n_acc = ((n + 1) + NS * 8 - 1) // (NS * 8) * (NS * 8)
    per_round = NS * BIG * IDXB
    e_pad = (e + per_round - 1) // per_round * per_round
    bpt = e_pad // (NS * BIG)

    src = edge_index[0]
    dst = edge_index[1]
    pad = e_pad - e
    src_p = jnp.concatenate([src, jnp.zeros((pad,), jnp.int32)])
    dst_p = jnp.concatenate([dst, jnp.full((pad,), n, jnp.int32)])
    src_big = src_p
    dst_big = dst_p.reshape(e_pad // CHUNK, CHUNK)
    ones_cols = jnp.ones((n_acc, 16), jnp.float32)

    deg_cpt = e_pad // (NT * CHUNK)
    degp = _sc_degree(dst_big, ones_cols, n_acc, deg_cpt)

    g1a, g1b, dinv8 = _tc_pre(x, W1, degp, n_acc)
    p1 = _sc_aggregate(g1a, g1b, src_big, dst_big, n_acc, dh, bpt)
    g2a, g2b = _tc_mid(dinv8, p1, b1.reshape(1, -1), W2)
    p2 = _sc_aggregate(g2a, g2b, src_big, dst_big, n_acc, dh, bpt)
    g3a, g3b = _tc_mid(dinv8, p2, b2.reshape(1, -1), W3)
    p3 = _sc_aggregate(g3a, g3b, src_big, dst_big, n_acc, dh, bpt)
    return _tc_final(dinv8, p3, b3.reshape(1, -1), n)


# speedup vs baseline: 1.0565x; 1.0565x over previous
import functools

import jax
import jax.numpy as jnp
from jax import lax
from jax.experimental import pallas as pl
from jax.experimental.pallas import tpu as pltpu
from jax.experimental.pallas import tpu_sc as plsc

NC = 2
NS = 16
NT = NC * NS
CHUNK = 128
BIG = 256
IDXB = 8


def _mesh():
    return plsc.VectorSubcoreMesh(
        core_axis_name="c", subcore_axis_name="s", num_cores=NC, num_subcores=NS
    )


def _sc_degree(dst_chunks, ones_cols, n_acc, cpt):
    rps = n_acc // NS
    DW = CHUNK

    @functools.partial(
        pl.kernel,
        out_type=jax.ShapeDtypeStruct((NC, n_acc, DW), jnp.float32),
        mesh=_mesh(),
        scratch_types=[
            pltpu.VMEM((cpt, CHUNK), jnp.int32),
            pltpu.VMEM((CHUNK, DW), jnp.float32),
            pltpu.VMEM_SHARED((n_acc, DW), jnp.float32),
        ],
    )
    def deg_kernel(dst_hbm, ones_hbm, out_hbm, dst_v, ones_v, acc):
        c = lax.axis_index("c")
        s = lax.axis_index("s")
        tile = c * NS + s
        pltpu.sync_copy(dst_hbm.at[pl.ds(tile * cpt, cpt)], dst_v)
        pltpu.sync_copy(ones_hbm.at[pl.ds(0, CHUNK)], ones_v)
        pltpu.sync_copy(
            ones_hbm.at[pl.ds(s * rps, rps)], acc.at[pl.ds(s * rps, rps)]
        )
        plsc.subcore_barrier()

        @pl.loop(0, cpt)
        def _(j):
            pltpu.sync_copy(ones_v, acc.at[dst_v.at[j]], add=True)

        plsc.subcore_barrier()
        pltpu.sync_copy(
            acc.at[pl.ds(s * rps, rps)], out_hbm.at[c].at[pl.ds(s * rps, rps)]
        )

    return deg_kernel(dst_chunks, ones_cols)


def _sc_aggregate(g_a, g_b, src_big, dst_big, n_acc, dh, bpt):
    rps = n_acc // NS

    @functools.partial(
        pl.kernel,
        out_type=jax.ShapeDtypeStruct((NC, n_acc, dh), jnp.float32),
        mesh=_mesh(),
        scratch_types=[
            pltpu.VMEM((IDXB * BIG,), jnp.int32),
            pltpu.VMEM((IDXB * 2, CHUNK), jnp.int32),
            pltpu.VMEM((BIG, dh), jnp.float32),
            pltpu.VMEM((BIG, dh), jnp.float32),
            pltpu.VMEM_SHARED((n_acc, dh), jnp.float32),
            pltpu.SemaphoreType.DMA,
            pltpu.SemaphoreType.DMA,
            pltpu.SemaphoreType.DMA,
            pltpu.SemaphoreType.DMA,
        ],
        compiler_params=pltpu.CompilerParams(use_tc_tiling_on_sc=False),
    )
    def agg_kernel(
        ga_hbm, gb_hbm, src_hbm, dst_hbm, out_hbm, src_b, dst_b, rows0, rows1,
        acc, sem0, sem1, ssem0, ssem1,
    ):
        c = lax.axis_index("c")
        s = lax.axis_index("s")

        def run(tab_hbm):
            pltpu.sync_copy(
                tab_hbm.at[pl.ds(s * rps, rps)], acc.at[pl.ds(s * rps, rps)]
            )
            plsc.subcore_barrier()

            @pl.loop(0, bpt // IDXB)
            def _(blk):
                base = s * bpt + blk * IDXB
                pltpu.sync_copy(
                    src_hbm.at[pl.ds(base * BIG, IDXB * BIG)], src_b
                )
                pltpu.sync_copy(dst_hbm.at[pl.ds(base * 2, IDXB * 2)], dst_b)
                pltpu.async_copy(
                    tab_hbm.at[src_b.at[pl.ds(0, BIG)]], rows0, sem0
                )

                def wait_scats(rows, ssem):
                    pltpu.make_async_copy(
                        rows.at[pl.ds(0, CHUNK)], acc.at[dst_b.at[0]], ssem
                    ).wait()
                    pltpu.make_async_copy(
                        rows.at[pl.ds(CHUNK, CHUNK)], acc.at[dst_b.at[0]], ssem
                    ).wait()

                @pl.loop(0, IDXB, step=2)
                def _(j):
                    pltpu.make_async_copy(
                        tab_hbm.at[src_b.at[pl.ds(j * BIG, BIG)]], rows0, sem0
                    ).wait()

                    @pl.when(j > 0)
                    def _():
                        wait_scats(rows1, ssem1)

                    pltpu.async_copy(
                        tab_hbm.at[src_b.at[pl.ds((j + 1) * BIG, BIG)]],
                        rows1, sem1,
                    )
                    pltpu.async_copy(
                        rows0.at[pl.ds(0, CHUNK)], acc.at[dst_b.at[2 * j]],
                        ssem0, add=True,
                    )
                    pltpu.async_copy(
                        rows0.at[pl.ds(CHUNK, CHUNK)],
                        acc.at[dst_b.at[2 * j + 1]], ssem0, add=True,
                    )
                    pltpu.make_async_copy(
                        tab_hbm.at[src_b.at[pl.ds(0, BIG)]], rows1, sem1
                    ).wait()

                    @pl.when(j + 2 < IDXB)
                    def _():
                        wait_scats(rows0, ssem0)
                        pltpu.async_copy(
                            tab_hbm.at[src_b.at[pl.ds((j + 2) * BIG, BIG)]],
                            rows0, sem0,
                        )

                    pltpu.async_copy(
                        rows1.at[pl.ds(0, CHUNK)],
                        acc.at[dst_b.at[2 * j + 2]], ssem1, add=True,
                    )
                    pltpu.async_copy(
                        rows1.at[pl.ds(CHUNK, CHUNK)],
                        acc.at[dst_b.at[2 * j + 3]], ssem1, add=True,
                    )

                wait_scats(rows0, ssem0)
                wait_scats(rows1, ssem1)

            plsc.subcore_barrier()
            pltpu.sync_copy(
                acc.at[pl.ds(s * rps, rps)],
                out_hbm.at[c].at[pl.ds(s * rps, rps)],
            )

        @pl.when(c == 0)
        def _():
            run(ga_hbm)

        @pl.when(c == 1)
        def _():
            run(gb_hbm)

    return agg_kernel(g_a, g_b, src_big, dst_big)


def _dinv(degp_ref):
    deg = degp_ref[0, :, 0:1] + degp_ref[1, :, 0:1] - 1.0
    return lax.rsqrt(jnp.maximum(deg, 1.0))


_DOT = dict(preferred_element_type=jnp.float32, precision=lax.Precision.HIGHEST)


def _tc_pre(x, w, degp, n_acc):
    n, _ = x.shape
    d = w.shape[1]
    dh = d // 2

    def body(x_ref, w_ref, degp_ref, ga_ref, gb_ref, dinv_ref):
        dinv = _dinv(degp_ref)
        h = jnp.dot(x_ref[...], w_ref[...], **_DOT)
        g = dinv[:n] * h
        pad = jnp.zeros((n_acc - n, dh), jnp.float32)
        ga_ref[...] = jnp.concatenate([g[:, :dh], pad], axis=0)
        gb_ref[...] = jnp.concatenate([g[:, dh:], pad], axis=0)
        dinv_ref[...] = jnp.broadcast_to(dinv, (n_acc, 8))

    return pl.pallas_call(
        body,
        out_shape=[
            jax.ShapeDtypeStruct((n_acc, dh), jnp.float32),
            jax.ShapeDtypeStruct((n_acc, dh), jnp.float32),
            jax.ShapeDtypeStruct((n_acc, 8), jnp.float32),
        ],
    )(x, w, degp)


def _tc_mid(dinv8, part, b, w_next):
    n_acc = part.shape[1]
    d = w_next.shape[1]
    dh = d // 2

    def body(dinv_ref, p_ref, b_ref, w_ref, ga_ref, gb_ref):
        dinv = dinv_ref[:, 0:1]
        agg = jnp.concatenate([p_ref[0], p_ref[1]], axis=1)
        x_next = jnp.maximum(dinv * agg + b_ref[...], 0.0)
        g = dinv * jnp.dot(x_next, w_ref[...], **_DOT)
        ga_ref[...] = g[:, :dh]
        gb_ref[...] = g[:, dh:]

    return pl.pallas_call(
        body,
        out_shape=[
            jax.ShapeDtypeStruct((n_acc, dh), jnp.float32),
            jax.ShapeDtypeStruct((n_acc, dh), jnp.float32),
        ],
    )(dinv8, part, b, w_next)


def _tc_final(dinv8, part, b, n):
    n_acc = part.shape[1]
    d = part.shape[2] * 2

    def body(dinv_ref, p_ref, b_ref, o_ref):
        dinv = dinv_ref[:, 0:1]
        agg = jnp.concatenate([p_ref[0], p_ref[1]], axis=1)
        o_ref[...] = (dinv * agg + b_ref[...])[:n]

    return pl.pallas_call(
        body, out_shape=jax.ShapeDtypeStruct((n, d), jnp.float32)
    )(dinv8, part, b)


def kernel(x, edge_index, W1, b1, W2, b2, W3, b3):
    n = x.shape[0]
    e = edge_index.shape[1]
    d = W1.shape[1]
    dh = d // 2

    n_acc = ((n + 1) + NS * 8 - 1) // (NS * 8) * (NS * 8)
    per_round = NS * BIG * IDXB
    e_pad = (e + per_round - 1) // per_round * per_round
    bpt = e_pad // (NS * BIG)

    src = edge_index[0]
    dst = edge_index[1]
    pad = e_pad - e
    src_p = jnp.concatenate([src, jnp.zeros((pad,), jnp.int32)])
    dst_p = jnp.concatenate([dst, jnp.full((pad,), n, jnp.int32)])
    src_big = src_p
    dst_big = dst_p.reshape(e_pad // CHUNK, CHUNK)
    ones_cols = jnp.ones((n_acc, CHUNK), jnp.float32)

    deg_cpt = e_pad // (NT * CHUNK)
    degp = _sc_degree(dst_big, ones_cols, n_acc, deg_cpt)

    g1a, g1b, dinv8 = _tc_pre(x, W1, degp, n_acc)
    p1 = _sc_aggregate(g1a, g1b, src_big, dst_big, n_acc, dh, bpt)
    g2a, g2b = _tc_mid(dinv8, p1, b1.reshape(1, -1), W2)
    p2 = _sc_aggregate(g2a, g2b, src_big, dst_big, n_acc, dh, bpt)
    g3a, g3b = _tc_mid(dinv8, p2, b2.reshape(1, -1), W3)
    p3 = _sc_aggregate(g3a, g3b, src_big, dst_big, n_acc, dh, bpt)
    return _tc_final(dinv8, p3, b3.reshape(1, -1), n)
